# baseline (speedup 1.0000x reference)
import jax
import jax.numpy as jnp
from jax import lax
from jax.experimental import pallas as pl

B, C, D = 1024, 512, 64
BB = 16


def _body(x_ref, masked_ref, idx_ref):
    x = x_ref[...]
    s = jnp.sum(x * x, axis=2)
    smax = jnp.max(s, axis=1, keepdims=True)
    c_iota = lax.broadcasted_iota(jnp.int32, (BB, C), 1)
    idx = jnp.min(jnp.where(s >= smax, c_iota, C), axis=1, keepdims=True)
    g_iota = lax.broadcasted_iota(jnp.int32, (BB, BB * C), 1)
    b_iota = lax.broadcasted_iota(jnp.int32, (BB, BB * C), 0)
    onehot = (g_iota == idx + b_iota * C).astype(jnp.float32)
    masked = jnp.dot(onehot, x.reshape(BB * C, D),
                     preferred_element_type=jnp.float32)
    masked_ref[...] = masked
    idx_ref[...] = idx


@jax.jit
def kernel(x):
    grid = (B // BB,)
    masked, idx = pl.pallas_call(
        _body,
        grid=grid,
        in_specs=[pl.BlockSpec((BB, C, D), lambda i: (i, 0, 0))],
        out_specs=[
            pl.BlockSpec((BB, D), lambda i: (i, 0)),
            pl.BlockSpec((BB, 1), lambda i: (i, 0)),
        ],
        out_shape=[
            jax.ShapeDtypeStruct((B, D), jnp.float32),
            jax.ShapeDtypeStruct((B, 1), jnp.int32),
        ],
    )(x)
    return masked[:, None, :], idx.reshape(B)
